# baseline (speedup 1.0000x reference)
import functools

import jax
import jax.numpy as jnp
from jax import lax
from jax.experimental import pallas as pl
from jax.experimental.pallas import tpu as pltpu
from jax.experimental.pallas import tpu_sc as plsc

_LANES = 16
_SUPER = 4
_SUB = 8
_CH = 64


def _rsqrt_newton(x):
    i = lax.bitcast_convert_type(x, jnp.int32)
    y = lax.bitcast_convert_type(
        jnp.full_like(i, 0x5F3759DF) - lax.shift_right_logical(i, 1),
        jnp.float32)
    for _ in range(3):
        y = y * (jnp.float32(1.5) - jnp.float32(0.5) * x * y * y)
    return y


def _lane_shuffle(x, idx):
    return lax.gather(
        x, idx[:, None],
        dimension_numbers=lax.GatherDimensionNumbers(
            offset_dims=(), collapsed_slice_dims=(0,), start_index_map=(0,)),
        slice_sizes=(1,),
        mode=lax.GatherScatterMode.PROMISE_IN_BOUNDS)


@functools.lru_cache(maxsize=None)
def _build(B, D, n_etiles, n_rtiles):
    info = plsc.get_sparse_core_info()
    nc, ns = info.num_cores, info.num_subcores
    nw = nc * ns
    b_per_w = B // nw
    n_ch = b_per_w // _CH
    nvec2 = D // (2 * _LANES)

    mesh = plsc.VectorSubcoreMesh(core_axis_name="c", subcore_axis_name="s")

    @functools.partial(
        pl.kernel,
        mesh=mesh,
        out_type=jax.ShapeDtypeStruct((B, D), jnp.float32),
        scratch_types=[
            pltpu.VMEM((b_per_w,), jnp.int32),
            pltpu.VMEM((b_per_w,), jnp.int32),
            pltpu.VMEM((2, _CH, 2 * D), jnp.int32),
            pltpu.VMEM((2, _CH, 2 * D), jnp.int32),
            pltpu.VMEM((_CH, D), jnp.float32),
            pltpu.SemaphoreType.DMA,
            pltpu.SemaphoreType.DMA,
        ],
    )
    def sc_kernel(heads_hbm, rels_hbm, etab_hbm, rtab_hbm, out_hbm,
                  hidx, ridx, erow, rrow, outbuf, gsem0, gsem1):
        wid = lax.axis_index("s") * nc + lax.axis_index("c")
        row_base = wid * b_per_w
        sems = (gsem0, gsem1)

        pltpu.sync_copy(heads_hbm.at[pl.ds(row_base, b_per_w)], hidx)
        pltpu.sync_copy(rels_hbm.at[pl.ds(row_base, b_per_w)], ridx)

        lanes = lax.iota(jnp.int32, _LANES)
        perms = [lanes ^ p for p in (8, 4, 2, 1)]
        himask = jnp.full((_LANES,), -65536, jnp.int32)
        half_lo = lax.shift_right_logical(lanes, 1)
        half_hi = half_lo + _LANES // 2
        even_mask = (lanes & 1) == 0

        def fire_chunk(c):
            par = c % 2
            sem = sems[par]
            cbase = c * _CH

            def fire(g, carry):
                base = cbase + g * _LANES
                vh = hidx[pl.ds(base, _LANES)]
                vr = ridx[pl.ds(base, _LANES)]
                vht = lax.shift_right_logical(vh, 5)
                vhs = lax.shift_right_logical(vh, 2) & (_SUB - 1)
                vrt = lax.shift_right_logical(vr, 5)
                vrs = lax.shift_right_logical(vr, 2) & (_SUB - 1)
                for l in range(_LANES):
                    j = g * _LANES + l
                    pltpu.async_copy(
                        etab_hbm.at[vht[l], vhs[l]], erow.at[par, j], sem)
                    pltpu.async_copy(
                        rtab_hbm.at[vrt[l], vrs[l]], rrow.at[par, j], sem)
                return carry

            lax.fori_loop(0, _CH // _LANES, fire, 0)

        dummy2d = etab_hbm.reshape(n_etiles * _SUB, 2 * D)

        def drain_chunk(c):
            par = c % 2
            sem = sems[par]
            pltpu.make_async_copy(
                dummy2d.at[pl.ds(0, _CH)], erow.at[par], sem).wait()
            pltpu.make_async_copy(
                dummy2d.at[pl.ds(0, _CH)], rrow.at[par], sem).wait()

        def compute_chunk(c):
            par = c % 2
            cbase = c * _CH

            def row_grp(g, carry):
                base = cbase + g * _LANES
                vhq = (hidx[pl.ds(base, _LANES)] & (_SUPER - 1)) * (D // 2)
                vrq = (ridx[pl.ds(base, _LANES)] & (_SUPER - 1)) * (D // 2)
                for l in range(_LANES):
                    j = g * _LANES + l
                    hq = vhq[l]
                    rq = vrq[l]
                    vs = []
                    ss = None
                    for k in range(nvec2):
                        ew = erow[par, j, pl.ds(hq + k * _LANES, _LANES)]
                        rw = rrow[par, j, pl.ds(rq + k * _LANES, _LANES)]
                        ea = lax.bitcast_convert_type(
                            lax.shift_left(ew, 16), jnp.float32)
                        eb = lax.bitcast_convert_type(
                            ew & himask, jnp.float32)
                        ra = lax.bitcast_convert_type(
                            lax.shift_left(rw, 16), jnp.float32)
                        rb = lax.bitcast_convert_type(
                            rw & himask, jnp.float32)
                        va = ea + ra
                        vb = eb + rb
                        vs.append((k, va, vb))
                        sq = va * va + vb * vb
                        ss = sq if ss is None else ss + sq
                    for p in perms:
                        ss = ss + _lane_shuffle(ss, p)
                    norm = ss * _rsqrt_newton(ss)
                    inv = jnp.float32(1.0) / jnp.maximum(
                        norm, jnp.float32(1e-12))
                    for k, va, vb in vs:
                        sa = va * inv
                        sb = vb * inv
                        lo = jnp.where(even_mask,
                                       _lane_shuffle(sa, half_lo),
                                       _lane_shuffle(sb, half_lo))
                        hi = jnp.where(even_mask,
                                       _lane_shuffle(sa, half_hi),
                                       _lane_shuffle(sb, half_hi))
                        outbuf[j, pl.ds(2 * k * _LANES, _LANES)] = lo
                        outbuf[j, pl.ds((2 * k + 1) * _LANES, _LANES)] = hi
                return carry

            lax.fori_loop(0, _CH // _LANES, row_grp, 0)
            pltpu.sync_copy(outbuf, out_hbm.at[pl.ds(row_base + cbase, _CH)])

        fire_chunk(0)
        for c in range(n_ch):
            if c + 1 < n_ch:
                fire_chunk(c + 1)
            drain_chunk(c)
            compute_chunk(c)

    return sc_kernel


def _pack_i32(tab, D):
    n = tab.shape[0]
    b = tab.astype(jnp.bfloat16).reshape(n, D // 2, 2)
    w = lax.bitcast_convert_type(b, jnp.int32)
    return w.reshape(n // (_SUPER * _SUB), _SUB, _SUPER * (D // 2))


def kernel(heads, relations, entity_table, relation_table):
    B = heads.shape[0]
    N, D = entity_table.shape
    R = relation_table.shape[0]
    grp = _SUPER * _SUB
    rpad = (-R) % grp
    rtab_p = jnp.concatenate(
        [relation_table, jnp.zeros((rpad, D), relation_table.dtype)]) \
        if rpad else relation_table
    etab3 = _pack_i32(entity_table, D)
    rtab3 = _pack_i32(rtab_p, D)
    fn = _build(B, D, etab3.shape[0], rtab3.shape[0])
    return fn(heads.astype(jnp.int32), relations.astype(jnp.int32),
              etab3, rtab3)


# speedup vs baseline: 6.9322x; 6.9322x over previous
import functools

import jax
import jax.numpy as jnp
from jax import lax
from jax.experimental import pallas as pl
from jax.experimental.pallas import tpu as pltpu
from jax.experimental.pallas import tpu_sc as plsc

_LANES = 16
_TILE = 8
_CH = 128


def _rsqrt_newton(x):
    i = lax.bitcast_convert_type(x, jnp.int32)
    y = lax.bitcast_convert_type(
        jnp.full_like(i, 0x5F3759DF) - lax.shift_right_logical(i, 1),
        jnp.float32)
    for _ in range(3):
        y = y * (jnp.float32(1.5) - jnp.float32(0.5) * x * y * y)
    return y


def _lane_shuffle(x, idx):
    return lax.gather(
        x, idx[:, None],
        dimension_numbers=lax.GatherDimensionNumbers(
            offset_dims=(), collapsed_slice_dims=(0,), start_index_map=(0,)),
        slice_sizes=(1,),
        mode=lax.GatherScatterMode.PROMISE_IN_BOUNDS)


@functools.lru_cache(maxsize=None)
def _build(B, D, n_etiles, n_rtiles):
    info = plsc.get_sparse_core_info()
    nc, ns = info.num_cores, info.num_subcores
    nw = nc * ns
    b_per_w = B // nw
    n_ch = b_per_w // _CH
    nvec = D // _LANES

    mesh = plsc.VectorSubcoreMesh(core_axis_name="c", subcore_axis_name="s")

    @functools.partial(
        pl.kernel,
        mesh=mesh,
        out_type=jax.ShapeDtypeStruct((B, D), jnp.float32),
        scratch_types=[
            pltpu.VMEM((b_per_w,), jnp.int32),
            pltpu.VMEM((b_per_w,), jnp.int32),
            pltpu.VMEM((2, _CH, D), jnp.float32),
            pltpu.VMEM((2, _CH, D), jnp.float32),
            pltpu.VMEM((2, _CH, D), jnp.float32),
            pltpu.SemaphoreType.DMA,
            pltpu.SemaphoreType.DMA,
            pltpu.SemaphoreType.DMA,
            pltpu.SemaphoreType.DMA,
        ],
    )
    def sc_kernel(heads_hbm, rels_hbm, etab_hbm, rtab_hbm, out_hbm,
                  hidx, ridx, erow, rrow, outbuf, gsem0, gsem1,
                  osem0, osem1):
        wid = lax.axis_index("s") * nc + lax.axis_index("c")
        row_base = wid * b_per_w
        sems = (gsem0, gsem1)
        osems = (osem0, osem1)

        pltpu.sync_copy(heads_hbm.at[pl.ds(row_base, b_per_w)], hidx)
        pltpu.sync_copy(rels_hbm.at[pl.ds(row_base, b_per_w)], ridx)

        lanes = lax.iota(jnp.int32, _LANES)
        perms = [lanes ^ p for p in (8, 4, 2, 1)]

        def fire_chunk(c):
            par = c % 2
            sem = sems[par]
            cbase = c * _CH

            def fire(g, carry):
                base = cbase + g * _LANES
                vh = hidx[pl.ds(base, _LANES)]
                vr = ridx[pl.ds(base, _LANES)]
                vht = lax.shift_right_logical(vh, 3)
                vhr = vh & (_TILE - 1)
                vrt = lax.shift_right_logical(vr, 3)
                vrr = vr & (_TILE - 1)
                for l in range(_LANES):
                    j = g * _LANES + l
                    pltpu.async_copy(
                        etab_hbm.at[vht[l], vhr[l]], erow.at[par, j], sem)
                    pltpu.async_copy(
                        rtab_hbm.at[vrt[l], vrr[l]], rrow.at[par, j], sem)
                return carry

            lax.fori_loop(0, _CH // _LANES, fire, 0)

        def drain_chunk(c):
            par = c % 2
            sem = sems[par]
            pltpu.make_async_copy(
                out_hbm.at[pl.ds(0, _CH)], erow.at[par], sem).wait()
            pltpu.make_async_copy(
                out_hbm.at[pl.ds(0, _CH)], rrow.at[par], sem).wait()

        def compute_chunk(c):
            par = c % 2
            cbase = c * _CH

            def row_fn(j, carry):
                vs = []
                ss = None
                for k in range(nvec):
                    sl = pl.ds(k * _LANES, _LANES)
                    v = erow[par, j, sl] + rrow[par, j, sl]
                    vs.append(v)
                    sq = v * v
                    ss = sq if ss is None else ss + sq
                for p in perms:
                    ss = ss + _lane_shuffle(ss, p)
                norm = ss * _rsqrt_newton(ss)
                inv = jnp.float32(1.0) / jnp.maximum(norm, jnp.float32(1e-12))
                for k, v in enumerate(vs):
                    outbuf[par, j, pl.ds(k * _LANES, _LANES)] = v * inv
                return carry

            lax.fori_loop(0, _CH, row_fn, 0)
            pltpu.async_copy(
                outbuf.at[par], out_hbm.at[pl.ds(row_base + cbase, _CH)],
                osems[par])

        def drain_out(c):
            par = c % 2
            pltpu.make_async_copy(
                outbuf.at[par], out_hbm.at[pl.ds(0, _CH)], osems[par]).wait()

        fire_chunk(0)
        for c in range(n_ch):
            if c + 1 < n_ch:
                fire_chunk(c + 1)
            drain_chunk(c)
            if c >= 2:
                drain_out(c - 2)
            compute_chunk(c)
        drain_out(n_ch - 2)
        drain_out(n_ch - 1)

    return sc_kernel


def kernel(heads, relations, entity_table, relation_table):
    B = heads.shape[0]
    N, D = entity_table.shape
    R = relation_table.shape[0]
    etab3 = entity_table.reshape(N // _TILE, _TILE, D)
    rtab3 = relation_table.reshape(R // _TILE, _TILE, D)
    fn = _build(B, D, N // _TILE, R // _TILE)
    return fn(heads.astype(jnp.int32), relations.astype(jnp.int32),
              etab3, rtab3)
